# baseline (device time: 27112 ns/iter reference)
import jax
import jax.numpy as jnp
from jax import lax
from jax.experimental import pallas as pl
from jax.experimental.pallas import tpu as pltpu

N_DEV = 32


def kernel(x, w_mat, scale_x, scale_w):
    m_global, k_per = x.shape
    k_global, n = w_mat.shape
    m_per = m_global // N_DEV

    def body(x_ref, w_ref, sx_ref, sw_ref, out_ref, xg_ref, send_sems, recv_sems):
        my = lax.axis_index("i")

        barrier = pltpu.get_barrier_semaphore()
        for k in range(1, N_DEV):
            peer = (my + k) % N_DEV
            pl.semaphore_signal(
                barrier, inc=1,
                device_id=(peer,), device_id_type=pl.DeviceIdType.MESH,
            )
        pl.semaphore_wait(barrier, N_DEV - 1)

        sends = []
        for k in range(1, N_DEV):
            peer = (my + k) % N_DEV
            rdma = pltpu.make_async_remote_copy(
                src_ref=x_ref.at[pl.ds(peer * m_per, m_per), :],
                dst_ref=xg_ref.at[my],
                send_sem=send_sems.at[peer],
                recv_sem=recv_sems.at[my],
                device_id=(peer,),
                device_id_type=pl.DeviceIdType.MESH,
            )
            rdma.start()
            sends.append(rdma)

        acc = lax.dot_general(
            x_ref[pl.ds(my * m_per, m_per), :],
            w_ref[pl.ds(my * k_per, k_per), :],
            (((1,), (0,)), ((), ())),
            preferred_element_type=jnp.int32,
        )

        for k in range(1, N_DEV):
            src = (my - k) % N_DEV
            recv = pltpu.make_async_remote_copy(
                src_ref=x_ref.at[pl.ds(src * m_per, m_per), :],
                dst_ref=xg_ref.at[src],
                send_sem=send_sems.at[src],
                recv_sem=recv_sems.at[src],
                device_id=(src,),
                device_id_type=pl.DeviceIdType.MESH,
            )
            recv.wait_recv()
            acc = acc + lax.dot_general(
                xg_ref[src],
                w_ref[pl.ds(src * k_per, k_per), :],
                (((1,), (0,)), ((), ())),
                preferred_element_type=jnp.int32,
            )

        out_ref[:, :] = acc.astype(jnp.float32) * (sx_ref[0] * sw_ref[0])

        for rdma in sends:
            rdma.wait_send()

    return pl.pallas_call(
        body,
        out_shape=jax.ShapeDtypeStruct((m_per, n), jnp.float32),
        in_specs=[
            pl.BlockSpec(memory_space=pltpu.VMEM),
            pl.BlockSpec(memory_space=pltpu.VMEM),
            pl.BlockSpec(memory_space=pltpu.SMEM),
            pl.BlockSpec(memory_space=pltpu.SMEM),
        ],
        out_specs=pl.BlockSpec(memory_space=pltpu.VMEM),
        scratch_shapes=[
            pltpu.VMEM((N_DEV, m_per, k_per), jnp.int8),
            pltpu.SemaphoreType.DMA((N_DEV,)),
            pltpu.SemaphoreType.DMA((N_DEV,)),
        ],
        compiler_params=pltpu.CompilerParams(collective_id=0),
    )(x, w_mat, scale_x, scale_w)


# device time: 24403 ns/iter; 1.1110x vs baseline; 1.1110x over previous
import jax
import jax.numpy as jnp
from jax import lax
from jax.experimental import pallas as pl
from jax.experimental.pallas import tpu as pltpu

N_DEV = 32
N_GROUPS = 4
G = N_DEV // N_GROUPS


def kernel(x, w_mat, scale_x, scale_w):
    m_global, k_per = x.shape
    k_global, n = w_mat.shape
    m_per = m_global // N_DEV

    def body(x_ref, w_ref, sx_ref, sw_ref, out_ref, xg_ref, send_sems, recv_sems):
        my = lax.axis_index("i")

        barrier = pltpu.get_barrier_semaphore()
        for k in range(1, N_DEV):
            peer = (my + k) % N_DEV
            pl.semaphore_signal(
                barrier, inc=1,
                device_id=(peer,), device_id_type=pl.DeviceIdType.MESH,
            )
        pl.semaphore_wait(barrier, N_DEV - 1)

        sends = []
        for k in range(N_DEV):
            peer = (my + k) % N_DEV
            rdma = pltpu.make_async_remote_copy(
                src_ref=x_ref.at[pl.ds(peer * m_per, m_per), :],
                dst_ref=xg_ref.at[:, pl.ds(my * k_per, k_per)],
                send_sem=send_sems.at[peer],
                recv_sem=recv_sems.at[my],
                device_id=(peer,),
                device_id_type=pl.DeviceIdType.MESH,
            )
            rdma.start()
            sends.append(rdma)

        acc = None
        for g in range(N_GROUPS):
            for s in range(g * G, (g + 1) * G):
                recv = pltpu.make_async_remote_copy(
                    src_ref=x_ref.at[pl.ds(s * m_per, m_per), :],
                    dst_ref=xg_ref.at[:, pl.ds(s * k_per, k_per)],
                    send_sem=send_sems.at[s],
                    recv_sem=recv_sems.at[s],
                    device_id=(s,),
                    device_id_type=pl.DeviceIdType.MESH,
                )
                recv.wait_recv()
            part = lax.dot_general(
                xg_ref[:, g * G * k_per:(g + 1) * G * k_per],
                w_ref[g * G * k_per:(g + 1) * G * k_per, :],
                (((1,), (0,)), ((), ())),
                preferred_element_type=jnp.int32,
            )
            acc = part if acc is None else acc + part

        out_ref[:, :] = acc.astype(jnp.float32) * (sx_ref[0] * sw_ref[0])

        for rdma in sends:
            rdma.wait_send()

    return pl.pallas_call(
        body,
        out_shape=jax.ShapeDtypeStruct((m_per, n), jnp.float32),
        in_specs=[
            pl.BlockSpec(memory_space=pltpu.VMEM),
            pl.BlockSpec(memory_space=pltpu.VMEM),
            pl.BlockSpec(memory_space=pltpu.SMEM),
            pl.BlockSpec(memory_space=pltpu.SMEM),
        ],
        out_specs=pl.BlockSpec(memory_space=pltpu.VMEM),
        scratch_shapes=[
            pltpu.VMEM((m_per, k_global), jnp.int8),
            pltpu.SemaphoreType.DMA((N_DEV,)),
            pltpu.SemaphoreType.DMA((N_DEV,)),
        ],
        compiler_params=pltpu.CompilerParams(collective_id=0),
    )(x, w_mat, scale_x, scale_w)
